# Initial kernel scaffold; baseline (speedup 1.0000x reference)
import jax
import jax.numpy as jnp
from jax.experimental import pallas as pl


def kernel(x, edge_index, W1_l, W1_r, b1, W2_l, W2_r, b2):
    raise NotImplementedError("write your pallas kernel here")


# speedup vs baseline: 4.5086x; 4.5086x over previous
import functools

import jax
import jax.numpy as jnp
from jax import lax
from jax.experimental import pallas as pl
from jax.experimental.pallas import tpu as pltpu
from jax.experimental.pallas import tpu_sc as plsc

N_NODES = 10000
N_EDGES = 320000
D = 128
DH = D // 2

NC = 2
NS = 16
E_PER_TILE = N_EDGES // NS
CHUNK = 128
NFULL = E_PER_TILE // CHUNK
TAIL = E_PER_TILE - NFULL * CHUNK
ROWS_PER_TILE = 624
ZROWS = 208
EXTRA_ROWS = N_NODES - NS * ROWS_PER_TILE
DEG_W = 16


def _zero_vmem(buf, nrows, ncols):
    z16 = jnp.zeros((16,), jnp.float32)

    def row(r, carry):
        for j in range(ncols // 16):
            buf[r, pl.ds(j * 16, 16)] = z16
        return carry

    lax.fori_loop(0, nrows, row, 0)


def _fill_ones(buf, nrows):
    o16 = jnp.ones((16,), jnp.float32)

    def row(r, carry):
        buf[r, :] = o16
        return carry

    lax.fori_loop(0, nrows, row, 0)


def _sc_agg_body(with_deg, *refs):
    if with_deg:
        (x0_hbm, x1_hbm, src_hbm, dst_hbm, agg_out, deg_out,
         zbuf, zdbuf, rows_v, rows_t, ones_v, ones_t,
         sidx, didx, sidx_t, didx_t, acc_sh, deg_sh, sem) = refs
    else:
        (x0_hbm, x1_hbm, src_hbm, dst_hbm, agg_out,
         zbuf, rows_v, rows_t,
         sidx, didx, sidx_t, didx_t, acc_sh, sem) = refs

    c = lax.axis_index("c")
    s = lax.axis_index("s")
    is_sc0 = c == 0
    last_tile = s == NS - 1
    extra0 = NS * ROWS_PER_TILE

    _zero_vmem(zbuf, ZROWS, DH)
    for b in range(ROWS_PER_TILE // ZROWS):
        r0 = s * ROWS_PER_TILE + b * ZROWS
        pltpu.sync_copy(zbuf, acc_sh.at[pl.ds(r0, ZROWS)])

    @pl.when(last_tile)
    def _():
        pltpu.sync_copy(zbuf.at[pl.ds(0, EXTRA_ROWS)],
                        acc_sh.at[pl.ds(extra0, EXTRA_ROWS)])

    if with_deg:
        _zero_vmem(zdbuf, ZROWS, DEG_W)
        _fill_ones(ones_v, CHUNK)
        _fill_ones(ones_t, TAIL)
        for b in range(ROWS_PER_TILE // ZROWS):
            r0 = s * ROWS_PER_TILE + b * ZROWS
            pltpu.sync_copy(zdbuf, deg_sh.at[pl.ds(r0, ZROWS)])

        @pl.when(last_tile)
        def _():
            pltpu.sync_copy(zdbuf.at[pl.ds(0, EXTRA_ROWS)],
                            deg_sh.at[pl.ds(extra0, EXTRA_ROWS)])

    plsc.subcore_barrier()

    e_base = s * E_PER_TILE

    def do_chunk(e0, rows, n_sidx, n_didx, n_ones):
        pltpu.sync_copy(src_hbm.at[pl.ds(e0, rows.shape[0])], n_sidx)
        pltpu.sync_copy(dst_hbm.at[pl.ds(e0, rows.shape[0])], n_didx)

        @pl.when(is_sc0)
        def _():
            pltpu.async_copy(x0_hbm.at[n_sidx], rows, sem).wait()

        @pl.when(jnp.logical_not(is_sc0))
        def _():
            pltpu.async_copy(x1_hbm.at[n_sidx], rows, sem).wait()

        pltpu.sync_copy(rows, acc_sh.at[n_didx], add=True)
        if with_deg:
            @pl.when(is_sc0)
            def _():
                pltpu.sync_copy(n_ones, deg_sh.at[n_didx], add=True)

    def chunk(i, carry):
        e0 = pl.multiple_of(e_base + i * CHUNK, 8)
        do_chunk(e0, rows_v, sidx, didx, ones_v if with_deg else None)
        return carry

    lax.fori_loop(0, NFULL, chunk, 0)

    e0 = pl.multiple_of(e_base + NFULL * CHUNK, 8)
    do_chunk(e0, rows_t, sidx_t, didx_t, ones_t if with_deg else None)

    plsc.subcore_barrier()

    r0 = s * ROWS_PER_TILE
    pltpu.sync_copy(acc_sh.at[pl.ds(r0, ROWS_PER_TILE)],
                    agg_out.at[c, pl.ds(r0, ROWS_PER_TILE)])

    @pl.when(last_tile)
    def _():
        pltpu.sync_copy(acc_sh.at[pl.ds(extra0, EXTRA_ROWS)],
                        agg_out.at[c, pl.ds(extra0, EXTRA_ROWS)])

    if with_deg:
        @pl.when(is_sc0)
        def _():
            pltpu.sync_copy(deg_sh.at[pl.ds(r0, ROWS_PER_TILE)],
                            deg_out.at[pl.ds(r0, ROWS_PER_TILE)])

        @pl.when(jnp.logical_and(is_sc0, last_tile))
        def _():
            pltpu.sync_copy(deg_sh.at[pl.ds(extra0, EXTRA_ROWS)],
                            deg_out.at[pl.ds(extra0, EXTRA_ROWS)])


@functools.lru_cache(maxsize=None)
def _sc_kernels():
    mesh = plsc.VectorSubcoreMesh(core_axis_name="c", subcore_axis_name="s",
                                  num_cores=NC, num_subcores=NS)
    idx_scratch = [
        pltpu.VMEM((CHUNK,), jnp.int32),
        pltpu.VMEM((CHUNK,), jnp.int32),
        pltpu.VMEM((TAIL,), jnp.int32),
        pltpu.VMEM((TAIL,), jnp.int32),
    ]
    sc_params = pltpu.CompilerParams(use_tc_tiling_on_sc=False)
    sc_agg_deg = pl.kernel(
        functools.partial(_sc_agg_body, True),
        compiler_params=sc_params,
        out_type=(
            jax.ShapeDtypeStruct((NC, N_NODES, DH), jnp.float32),
            jax.ShapeDtypeStruct((N_NODES, DEG_W), jnp.float32),
        ),
        mesh=mesh,
        scratch_types=[
            pltpu.VMEM((ZROWS, DH), jnp.float32),
            pltpu.VMEM((ZROWS, DEG_W), jnp.float32),
            pltpu.VMEM((CHUNK, DH), jnp.float32),
            pltpu.VMEM((TAIL, DH), jnp.float32),
            pltpu.VMEM((CHUNK, DEG_W), jnp.float32),
            pltpu.VMEM((TAIL, DEG_W), jnp.float32),
            *idx_scratch,
            pltpu.VMEM_SHARED((N_NODES, DH), jnp.float32),
            pltpu.VMEM_SHARED((N_NODES, DEG_W), jnp.float32),
            pltpu.SemaphoreType.DMA,
        ],
    )
    sc_agg = pl.kernel(
        functools.partial(_sc_agg_body, False),
        compiler_params=sc_params,
        out_type=jax.ShapeDtypeStruct((NC, N_NODES, DH), jnp.float32),
        mesh=mesh,
        scratch_types=[
            pltpu.VMEM((ZROWS, DH), jnp.float32),
            pltpu.VMEM((CHUNK, DH), jnp.float32),
            pltpu.VMEM((TAIL, DH), jnp.float32),
            *idx_scratch,
            pltpu.VMEM_SHARED((N_NODES, DH), jnp.float32),
            pltpu.SemaphoreType.DMA,
        ],
    )
    return sc_agg_deg, sc_agg


BLK = 400


def _mean_of(aggp, deg):
    agg = jnp.concatenate([aggp[0], aggp[1]], axis=-1)
    rdeg = 1.0 / jnp.maximum(deg[:, 0:1], 1.0)
    return agg * rdeg


def _dense1_body(aggp, deg, x, wl, wr, b, out0, out1):
    h = (jnp.dot(_mean_of(aggp, deg), wl[...],
                 preferred_element_type=jnp.float32)
         + jnp.dot(x[...], wr[...], preferred_element_type=jnp.float32)
         + b[...])
    h = jnp.maximum(h, 0.0)
    out0[...] = h[:, :DH]
    out1[...] = h[:, DH:]


def _dense2_body(aggp, deg, h0, h1, wl, wr, b, out):
    hin = jnp.concatenate([h0[...], h1[...]], axis=-1)
    out[...] = (jnp.dot(_mean_of(aggp, deg), wl[...],
                        preferred_element_type=jnp.float32)
                + jnp.dot(hin, wr[...], preferred_element_type=jnp.float32)
                + b[...])


_AGG_SPEC = pl.BlockSpec((NC, BLK, DH), lambda i: (0, i, 0))
_DEG_SPEC = pl.BlockSpec((BLK, DEG_W), lambda i: (i, 0))
_ROW_SPEC = pl.BlockSpec((BLK, D), lambda i: (i, 0))
_HALF_SPEC = pl.BlockSpec((BLK, DH), lambda i: (i, 0))
_W_SPEC = pl.BlockSpec((D, D), lambda i: (0, 0))
_B_SPEC = pl.BlockSpec((1, D), lambda i: (0, 0))


def _dense1(aggp, deg, x, wl, wr, b):
    return pl.pallas_call(
        _dense1_body,
        grid=(N_NODES // BLK,),
        in_specs=[_AGG_SPEC, _DEG_SPEC, _ROW_SPEC, _W_SPEC, _W_SPEC, _B_SPEC],
        out_specs=[_HALF_SPEC, _HALF_SPEC],
        out_shape=[jax.ShapeDtypeStruct((N_NODES, DH), jnp.float32)] * 2,
    )(aggp, deg, x, wl, wr, b)


def _dense2(aggp, deg, h0, h1, wl, wr, b):
    return pl.pallas_call(
        _dense2_body,
        grid=(N_NODES // BLK,),
        in_specs=[_AGG_SPEC, _DEG_SPEC, _HALF_SPEC, _HALF_SPEC,
                  _W_SPEC, _W_SPEC, _B_SPEC],
        out_specs=_ROW_SPEC,
        out_shape=jax.ShapeDtypeStruct((N_NODES, D), jnp.float32),
    )(aggp, deg, h0, h1, wl, wr, b)


def kernel(x, edge_index, W1_l, W1_r, b1, W2_l, W2_r, b2):
    sc_agg_deg, sc_agg = _sc_kernels()
    src = edge_index[0].astype(jnp.int32)
    dst = edge_index[1].astype(jnp.int32)
    x0 = x[:, :DH]
    x1 = x[:, DH:]
    agg1, deg = sc_agg_deg(x0, x1, src, dst)
    h0, h1 = _dense1(agg1, deg, x, W1_l, W1_r, b1.reshape(1, D))
    agg2 = sc_agg(h0, h1, src, dst)
    z = _dense2(agg2, deg, h0, h1, W2_l, W2_r, b2.reshape(1, D))
    return z
